# baseline (speedup 1.0000x reference)
import functools

import jax
import jax.numpy as jnp
from jax import lax
from jax.experimental import pallas as pl
from jax.experimental.pallas import tpu as pltpu
from jax.experimental.pallas import tpu_sc as plsc

NUM_CORES = 2
NUM_SUBCORES = 16
NUM_WORKERS = NUM_CORES * NUM_SUBCORES
LANES = 16

D = 64
NBLK = D // LANES
POS = 200
TOTAL_ROWS = 1024 * 200
ROWS_PER_WORKER = TOTAL_ROWS // NUM_WORKERS
CHUNK = 64
NCHUNKS = ROWS_PER_WORKER // CHUNK
NBUF = 5
NROUNDS = NCHUNKS // NBUF
NSTR = 4
EPS = 1e-05
UNROLL = 2


def _ln_body(ids_hbm, table_hbm, pos_hbm, gamma_hbm, beta_hbm, out_hbm,
             idx_v, in_v, res_v, pos_v, gamma_v, beta_v, *sems):
    gsem = sems[:NBUF * NSTR]
    ssem = sems[NBUF * NSTR:]
    wid = lax.axis_index("s") * NUM_CORES + lax.axis_index("c")
    worker_base = wid * ROWS_PER_WORKER

    pltpu.sync_copy(pos_hbm, pos_v)
    pltpu.sync_copy(gamma_hbm, gamma_v)
    pltpu.sync_copy(beta_hbm, beta_v)

    gammas = [gamma_v[pl.ds(blk * LANES, LANES)] for blk in range(NBLK)]
    betas = [beta_v[pl.ds(blk * LANES, LANES)] for blk in range(NBLK)]

    def start_fetch(c, b):
        pltpu.sync_copy(ids_hbm.at[pl.ds(worker_base + c * CHUNK, CHUNK)],
                        idx_v.at[b])

        def enqueue(g, carry):
            ids16 = idx_v[b, pl.ds(g * LANES, LANES)]
            for u in range(LANES):
                j = g * LANES + u
                pltpu.async_copy(table_hbm.at[ids16[u]], in_v.at[b, j],
                                 gsem[b * NSTR + (u % NSTR)])
            return carry

        lax.fori_loop(0, CHUNK // LANES, enqueue, 0)

    for b in range(NBUF):
        start_fetch(b, b)

    def round_body(cc, carry):
        for b in range(NBUF):
            c = cc * NBUF + b
            base = worker_base + c * CHUNK
            inb = in_v.at[b]
            resb = res_v.at[b]
            for q in range(NSTR):
                pltpu.make_async_copy(
                    out_hbm.at[pl.ds(0, CHUNK // NSTR)],
                    in_v.at[b, pl.ds(q * (CHUNK // NSTR), CHUNK // NSTR)],
                    gsem[b * NSTR + q]).wait()

            @pl.when(c >= NBUF)
            def _():
                pltpu.make_async_copy(out_hbm.at[pl.ds(0, CHUNK)], resb,
                                      ssem[b]).wait()

            iota16 = lax.iota(jnp.int32, LANES)

            def row_body(g, carry4):
                posrow = jnp.remainder(base + g * LANES + iota16, POS)
                for u in range(LANES):
                    j = g * LANES + u
                    pr = posrow[u]
                    w = [inb[j, pl.ds(blk * LANES, LANES)]
                         + pos_v[pr, pl.ds(blk * LANES, LANES)]
                         for blk in range(NBLK)]
                    tot = jnp.full((LANES,),
                                   jnp.sum(w[0] + w[1] + w[2] + w[3]))
                    sq = jnp.full(
                        (LANES,),
                        jnp.sum(w[0] * w[0] + w[1] * w[1]
                                + w[2] * w[2] + w[3] * w[3]))
                    mean = tot * (1.0 / D)
                    var = sq * (1.0 / D) - mean * mean
                    x = var + EPS
                    xi = plsc.bitcast(x, jnp.int32)
                    y = plsc.bitcast(jnp.int32(0x5F3759DF) - (xi >> 1),
                                     jnp.float32)
                    y = y * (1.5 - 0.5 * x * y * y)
                    y = y * (1.5 - 0.5 * x * y * y)
                    y = y * (1.5 - 0.5 * x * y * y)
                    for blk in range(NBLK):
                        resb[j, pl.ds(blk * LANES, LANES)] = (
                            (w[blk] - mean) * y * gammas[blk] + betas[blk])
                return carry4

            lax.fori_loop(0, CHUNK // LANES, row_body, 0)

            pltpu.async_copy(resb, out_hbm.at[pl.ds(base, CHUNK)], ssem[b])

            nxt = c + NBUF

            @pl.when(nxt < NCHUNKS)
            def _():
                start_fetch(nxt, b)

        return carry

    lax.fori_loop(0, NROUNDS, round_body, 0)

    for b in range(NBUF):
        pltpu.make_async_copy(out_hbm.at[pl.ds(0, CHUNK)], res_v.at[b],
                              ssem[b]).wait()


def kernel(input_ids, emb_table, pos_table, gamma, beta):
    batch, seq = input_ids.shape
    ids_flat = input_ids.reshape(batch * seq)
    mesh = plsc.VectorSubcoreMesh(
        core_axis_name="c", subcore_axis_name="s",
        num_cores=NUM_CORES, num_subcores=NUM_SUBCORES)
    run = functools.partial(
        pl.kernel,
        out_type=jax.ShapeDtypeStruct((TOTAL_ROWS, D), jnp.float32),
        mesh=mesh,
        compiler_params=pltpu.CompilerParams(
            needs_layout_passes=False, use_tc_tiling_on_sc=False),
        scratch_types=[
            pltpu.VMEM((NBUF, CHUNK), jnp.int32),
            pltpu.VMEM((NBUF, CHUNK, D), jnp.float32),
            pltpu.VMEM((NBUF, CHUNK, D), jnp.float32),
            pltpu.VMEM((POS, D), jnp.float32),
            pltpu.VMEM((D,), jnp.float32),
            pltpu.VMEM((D,), jnp.float32),
        ] + [pltpu.SemaphoreType.DMA] * (NBUF * NSTR + NBUF),
    )(_ln_body)
    out = run(ids_flat, emb_table, pos_table, gamma, beta)
    return out.reshape(batch, seq, D)


# speedup vs baseline: 1.0398x; 1.0398x over previous
import functools

import jax
import jax.numpy as jnp
from jax import lax
from jax.experimental import pallas as pl
from jax.experimental.pallas import tpu as pltpu
from jax.experimental.pallas import tpu_sc as plsc

NUM_CORES = 2
NUM_SUBCORES = 16
NUM_WORKERS = NUM_CORES * NUM_SUBCORES
LANES = 16

D = 64
NBLK = D // LANES
POS = 200
TOTAL_ROWS = 1024 * 200
ROWS_PER_WORKER = TOTAL_ROWS // NUM_WORKERS
CHUNK = 64
NCHUNKS = ROWS_PER_WORKER // CHUNK
NBUF = 5
NROUNDS = NCHUNKS // NBUF
EPS = 1e-05
UNROLL = 2


def _ln_body(ids_hbm, table_hbm, pos_hbm, gamma_hbm, beta_hbm, out_hbm,
             idx_v, in_v, res_v, pos_v, gamma_v, beta_v, *sems):
    gsem = sems[:NBUF]
    ssem = sems[NBUF:2 * NBUF]
    isem = sems[2 * NBUF:]
    wid = lax.axis_index("s") * NUM_CORES + lax.axis_index("c")
    worker_base = wid * ROWS_PER_WORKER

    pltpu.sync_copy(pos_hbm, pos_v)
    pltpu.sync_copy(gamma_hbm, gamma_v)
    pltpu.sync_copy(beta_hbm, beta_v)

    gammas = [gamma_v[pl.ds(blk * LANES, LANES)] for blk in range(NBLK)]
    betas = [beta_v[pl.ds(blk * LANES, LANES)] for blk in range(NBLK)]

    def start_idx(c, b):
        pltpu.async_copy(ids_hbm.at[pl.ds(worker_base + c * CHUNK, CHUNK)],
                         idx_v.at[b], isem[b])

    def wait_idx(b):
        pltpu.make_async_copy(ids_hbm.at[pl.ds(0, CHUNK)], idx_v.at[b],
                              isem[b]).wait()

    def start_gather(b):
        def enqueue(g, carry):
            ids16 = idx_v[b, pl.ds(g * LANES, LANES)]
            for u in range(LANES):
                j = g * LANES + u
                pltpu.async_copy(table_hbm.at[ids16[u]], in_v.at[b, j],
                                 gsem[b])
            return carry

        lax.fori_loop(0, CHUNK // LANES, enqueue, 0)

    for b in range(NBUF):
        start_idx(b, b)
    for b in range(NBUF):
        wait_idx(b)
        start_gather(b)
        start_idx(b + NBUF, b)

    def round_body(cc, carry):
        for b in range(NBUF):
            c = cc * NBUF + b
            base = worker_base + c * CHUNK
            inb = in_v.at[b]
            resb = res_v.at[b]
            pltpu.make_async_copy(out_hbm.at[pl.ds(0, CHUNK)], inb,
                                  gsem[b]).wait()

            @pl.when(c >= NBUF)
            def _():
                pltpu.make_async_copy(out_hbm.at[pl.ds(0, CHUNK)], resb,
                                      ssem[b]).wait()

            iota16 = lax.iota(jnp.int32, LANES)

            def row_body(g, carry4):
                posrow = jnp.remainder(base + g * LANES + iota16, POS)
                for u in range(LANES):
                    j = g * LANES + u
                    pr = posrow[u]
                    w = [inb[j, pl.ds(blk * LANES, LANES)]
                         + pos_v[pr, pl.ds(blk * LANES, LANES)]
                         for blk in range(NBLK)]
                    tot = jnp.full((LANES,),
                                   jnp.sum(w[0] + w[1] + w[2] + w[3]))
                    sq = jnp.full(
                        (LANES,),
                        jnp.sum(w[0] * w[0] + w[1] * w[1]
                                + w[2] * w[2] + w[3] * w[3]))
                    mean = tot * (1.0 / D)
                    var = sq * (1.0 / D) - mean * mean
                    x = var + EPS
                    xi = plsc.bitcast(x, jnp.int32)
                    y = plsc.bitcast(jnp.int32(0x5F3759DF) - (xi >> 1),
                                     jnp.float32)
                    y = y * (1.5 - 0.5 * x * y * y)
                    y = y * (1.5 - 0.5 * x * y * y)
                    y = y * (1.5 - 0.5 * x * y * y)
                    for blk in range(NBLK):
                        resb[j, pl.ds(blk * LANES, LANES)] = (
                            (w[blk] - mean) * y * gammas[blk] + betas[blk])
                return carry4

            lax.fori_loop(0, CHUNK // LANES, row_body, 0)

            pltpu.async_copy(resb, out_hbm.at[pl.ds(base, CHUNK)], ssem[b])

            nxt = c + NBUF

            @pl.when(nxt < NCHUNKS)
            def _():
                wait_idx(b)
                start_gather(b)

            @pl.when(nxt + NBUF < NCHUNKS)
            def _():
                start_idx(nxt + NBUF, b)

        return carry

    lax.fori_loop(0, NROUNDS, round_body, 0)

    for b in range(NBUF):
        pltpu.make_async_copy(out_hbm.at[pl.ds(0, CHUNK)], res_v.at[b],
                              ssem[b]).wait()


def kernel(input_ids, emb_table, pos_table, gamma, beta):
    batch, seq = input_ids.shape
    ids_flat = input_ids.reshape(batch * seq)
    mesh = plsc.VectorSubcoreMesh(
        core_axis_name="c", subcore_axis_name="s",
        num_cores=NUM_CORES, num_subcores=NUM_SUBCORES)
    run = functools.partial(
        pl.kernel,
        out_type=jax.ShapeDtypeStruct((TOTAL_ROWS, D), jnp.float32),
        mesh=mesh,
        compiler_params=pltpu.CompilerParams(
            needs_layout_passes=False, use_tc_tiling_on_sc=False),
        scratch_types=[
            pltpu.VMEM((NBUF, CHUNK), jnp.int32),
            pltpu.VMEM((NBUF, CHUNK, D), jnp.float32),
            pltpu.VMEM((NBUF, CHUNK, D), jnp.float32),
            pltpu.VMEM((POS, D), jnp.float32),
            pltpu.VMEM((D,), jnp.float32),
            pltpu.VMEM((D,), jnp.float32),
        ] + [pltpu.SemaphoreType.DMA] * (3 * NBUF),
    )(_ln_body)
    out = run(ids_flat, emb_table, pos_table, gamma, beta)
    return out.reshape(batch, seq, D)
